# baseline (speedup 1.0000x reference)
import math

import jax
import jax.numpy as jnp
from jax import lax
from jax.experimental import pallas as pl
from jax.experimental.pallas import tpu as pltpu
from jax.experimental.pallas import tpu_sc as plsc

L = 512
B = 8
NBINS = 24
NCUT = NBINS + 1

IB = 64
JB = 128
NI = L // IB
NJ = L // JB

EPS = 1e-6
EPS2 = 1e-12
STEP = 15.0 * math.pi / 180.0


def _stage1_body(cai_ref, cbi_ref, caj_ref, cbj_ref, maskf_ref, cut_ref, out_ref):
    ib = pl.program_id(0)
    jb = pl.program_id(1)
    @pl.when(jb * JB + (JB - 1) >= ib * IB + 1)
    def _compute():
        cuts = [cut_ref[0, k] for k in range(NCUT)]
        cbj = [cbj_ref[c] for c in range(3)]
        zc = [caj_ref[c] - cbj_ref[c] for c in range(3)]
        nz2 = zc[0] * zc[0] + zc[1] * zc[1] + zc[2] * zc[2]
        for ii in range(IB):
            xc = [cbi_ref[0, c, :, ii:ii + 1] - cai_ref[0, c, :, ii:ii + 1]
                  for c in range(3)]
            nx2 = xc[0] * xc[0] + xc[1] * xc[1] + xc[2] * xc[2]
            yc = [cbj[c] - cbi_ref[0, c, :, ii:ii + 1] for c in range(3)]
            ny2 = yc[0] * yc[0] + yc[1] * yc[1] + yc[2] * yc[2]
            ny = jnp.sqrt(ny2)
            cxy = [xc[1] * yc[2] - xc[2] * yc[1],
                   xc[2] * yc[0] - xc[0] * yc[2],
                   xc[0] * yc[1] - xc[1] * yc[0]]
            cyz = [yc[1] * zc[2] - yc[2] * zc[1],
                   yc[2] * zc[0] - yc[0] * zc[2],
                   yc[0] * zc[1] - yc[1] * zc[0]]
            cc = [cxy[1] * cyz[2] - cxy[2] * cyz[1],
                  cxy[2] * cyz[0] - cxy[0] * cyz[2],
                  cxy[0] * cyz[1] - cxy[1] * cyz[0]]
            sin = yc[0] * cc[0] + yc[1] * cc[1] + yc[2] * cc[2]
            cos = (cxy[0] * cyz[0] + cxy[1] * cyz[1] + cxy[2] * cyz[2]) * ny
            omega = jnp.arctan2(sin, cos)
            mrow = maskf_ref[0, ii:ii + 1, :] > 0.0
            good = jnp.logical_and(nx2 > EPS2, ny2 > EPS2)
            good = jnp.logical_and(good, nz2 > EPS2)
            good = jnp.logical_and(good, mrow)
            good = jnp.logical_and(good, sin * sin + cos * cos > EPS)
            ssum = jnp.zeros_like(omega)
            for k in range(NCUT):
                ssum = ssum + jnp.where(cuts[k] < omega, 1.0, 0.0)
            idxf = jnp.clip(ssum - 1.0, 0.0, float(NBINS - 1))
            xoff = omega - (cuts[0] + idxf * STEP)
            idslot = jnp.where(good, idxf, -1.0)
            packed = jnp.concatenate([xoff, idslot], axis=0)
            out_ref[ii] = packed.T


def _stage1(CAi, CBi, CAt, CBt, maskf, cutpad):
    return pl.pallas_call(
        _stage1_body,
        grid=(NI, NJ),
        in_specs=[
            pl.BlockSpec((1, 3, B, IB), lambda i, j: (i, 0, 0, 0)),
            pl.BlockSpec((1, 3, B, IB), lambda i, j: (i, 0, 0, 0)),
            pl.BlockSpec((3, B, JB), lambda i, j: (0, 0, j)),
            pl.BlockSpec((3, B, JB), lambda i, j: (0, 0, j)),
            pl.BlockSpec((1, IB, JB), lambda i, j: (i, 0, j)),
            pl.BlockSpec((1, 128), lambda i, j: (0, 0)),
        ],
        out_specs=pl.BlockSpec((IB, JB, 16), lambda i, j: (i, j, 0)),
        out_shape=jax.ShapeDtypeStruct((L, L, 16), jnp.float32),
    )(CAi, CBi, CAt, CBt, maskf, cutpad)


NC = 2
NS = 16
NW = NC * NS
RPW = L // NW


def _stage2_body(outtab_hbm, coeff_hbm, part_hbm, obuf, cbuf, accbuf, semO, semC):
    wid = lax.axis_index("s") * NC + lax.axis_index("c")
    lanes = lax.iota(jnp.int32, 16)

    def _row(r, acc):
        i = r * NW + wid
        jt_lo = (i + 1) // JB

        def _cell(jt, acc):
            cpo = pltpu.make_async_copy(
                outtab_hbm.at[pl.ds(i * L + jt * JB, JB)], obuf, semO)
            cpo.start()
            ccs = [pltpu.make_async_copy(coeff_hbm.at[i, bb, jt], cbuf.at[bb], semC)
                   for bb in range(NBINS)]
            for cp in ccs:
                cp.start()
            cpo.wait()
            for cp in ccs:
                cp.wait()
            for g in range(JB // 16):
                rowl = g * 16 + lanes
                for b in range(B):
                    idf = plsc.load_gather(
                        obuf, [rowl, jnp.full((16,), 8 + b, jnp.int32)])
                    xof = plsc.load_gather(
                        obuf, [rowl, jnp.full((16,), b, jnp.int32)])
                    mb = idf >= 0.0
                    bi = jnp.where(mb, idf, 0.0).astype(jnp.int32)
                    c0 = plsc.load_gather(cbuf, [bi, jnp.zeros((16,), jnp.int32), rowl])
                    c1 = plsc.load_gather(cbuf, [bi, jnp.full((16,), 1, jnp.int32), rowl])
                    c2 = plsc.load_gather(cbuf, [bi, jnp.full((16,), 2, jnp.int32), rowl])
                    c3 = plsc.load_gather(cbuf, [bi, jnp.full((16,), 3, jnp.int32), rowl])
                    val = ((c0 * xof + c1) * xof + c2) * xof + c3
                    acc = acc + jnp.where(mb, val, 0.0)
            return acc
        return lax.fori_loop(jt_lo, NJ, _cell, acc)
    acc = lax.fori_loop(0, RPW, _row, jnp.zeros((16,), jnp.float32))

    accbuf[...] = acc
    pltpu.sync_copy(accbuf, part_hbm.at[wid])


def _stage2(outtab, coeffp):
    mesh = plsc.VectorSubcoreMesh(core_axis_name="c", subcore_axis_name="s")
    f = pl.kernel(
        _stage2_body,
        out_type=jax.ShapeDtypeStruct((NW, 16), jnp.float32),
        mesh=mesh,
        scratch_types=[
            pltpu.VMEM((JB, 16), jnp.float32),
            pltpu.VMEM((NBINS, 4, JB), jnp.float32),
            pltpu.VMEM((16,), jnp.float32),
            pltpu.SemaphoreType.DMA,
            pltpu.SemaphoreType.DMA,
        ],
        compiler_params=pltpu.CompilerParams(
            needs_layout_passes=False, use_tc_tiling_on_sc=False),
    )
    return f(outtab, coeffp)


def kernel(CA, CB, mask, coeff, cutoffs):
    CAt = jnp.transpose(CA, (2, 0, 1))
    CBt = jnp.transpose(CB, (2, 0, 1))
    CAi = jnp.transpose(CAt.reshape(3, B, NI, IB), (2, 0, 1, 3))
    CBi = jnp.transpose(CBt.reshape(3, B, NI, IB), (2, 0, 1, 3))
    maskf = mask.astype(jnp.float32).reshape(NI, IB, L)
    cutpad = jnp.zeros((1, 128), jnp.float32).at[0, :NCUT].set(cutoffs)
    out = _stage1(CAi, CBi, CAt, CBt, maskf, cutpad)
    outtab = out.reshape(L * L, 16)
    coeffp = jnp.transpose(coeff.reshape(L, NJ, JB, NBINS, 4), (0, 3, 1, 4, 2))
    partials = _stage2(outtab, coeffp)
    return jnp.sum(partials)
